# baseline (speedup 1.0000x reference)
import functools

import jax
import jax.numpy as jnp
from jax import lax
from jax.experimental import pallas as pl
from jax.experimental.pallas import tpu as pltpu
from jax.experimental.pallas import tpu_sc as plsc

E = 8
TOP_K = 2
D = 1024
FF = 3584
TILE = 256

_INTERPRET = False


def _router_body(x_ref, g_ref, logits_ref, a0_ref, a1_ref, w0_ref, w1_ref):
    x = x_ref[...]
    logits = lax.dot_general(x, g_ref[...], (((1,), (1,)), ((), ())),
                             preferred_element_type=jnp.float32)
    logits_ref[...] = logits
    col = lax.broadcasted_iota(jnp.int32, logits.shape, 1)
    m0 = jnp.max(logits, axis=1, keepdims=True)
    is0 = logits == m0
    a0 = jnp.min(jnp.where(is0, col, E), axis=1, keepdims=True)
    masked = jnp.where(col == a0, -jnp.inf, logits)
    m1 = jnp.max(masked, axis=1, keepdims=True)
    a1 = jnp.min(jnp.where(masked == m1, col, E), axis=1, keepdims=True)
    a0_ref[...] = a0
    a1_ref[...] = a1
    w0_ref[...] = jax.nn.sigmoid(m0 - m1)
    w1_ref[...] = jax.nn.sigmoid(m1 - m0)


def _router(x, gate_w):
    T = x.shape[0]
    TB = 512
    grid = (T // TB,)
    out_shapes = (
        jax.ShapeDtypeStruct((T, E), jnp.float32),
        jax.ShapeDtypeStruct((T, 1), jnp.int32),
        jax.ShapeDtypeStruct((T, 1), jnp.int32),
        jax.ShapeDtypeStruct((T, 1), jnp.float32),
        jax.ShapeDtypeStruct((T, 1), jnp.float32),
    )
    row_spec = pl.BlockSpec((TB, 1), lambda i: (i, 0))
    return pl.pallas_call(
        _router_body,
        grid=grid,
        in_specs=[
            pl.BlockSpec((TB, D), lambda i: (i, 0)),
            pl.BlockSpec((E, D), lambda i: (0, 0)),
        ],
        out_specs=(pl.BlockSpec((TB, E), lambda i: (i, 0)),
                   row_spec, row_spec, row_spec, row_spec),
        out_shape=out_shapes,
        interpret=_INTERPRET,
    )(x, gate_w)


_NW = 16
_L = 16


def _sc_dispatch(a0, a1, x32, T, n_buf):
    tpw = T // _NW
    half = tpw // 2
    nchunk = tpw // _L
    nt = n_buf // TILE
    nt48 = 48
    mesh = plsc.VectorSubcoreMesh(core_axis_name="c", subcore_axis_name="s",
                                  num_cores=1)

    @functools.partial(
        pl.kernel, mesh=mesh,
        compiler_params=pltpu.CompilerParams(needs_layout_passes=False),
        out_type=[
            jax.ShapeDtypeStruct((_NW, 16), jnp.int32),
            jax.ShapeDtypeStruct((2 * _NW, half), jnp.int32),
            jax.ShapeDtypeStruct((2 * _NW, half), jnp.int32),
            jax.ShapeDtypeStruct((nt48,), jnp.int32),
            jax.ShapeDtypeStruct((nt48,), jnp.int32),
            jax.ShapeDtypeStruct((n_buf, x32.shape[1]), jnp.float32),
        ],
        scratch_types=[
            pltpu.VMEM((tpw,), jnp.int32),
            pltpu.VMEM((tpw,), jnp.int32),
            pltpu.VMEM((1, 16), jnp.int32),
            pltpu.VMEM((_NW, 16), jnp.int32),
            pltpu.VMEM((2, half), jnp.int32),
            pltpu.VMEM((2, half), jnp.int32),
            pltpu.VMEM((half, x32.shape[1]), jnp.float32),
            pltpu.VMEM((nt48,), jnp.int32),
            pltpu.VMEM((nt48,), jnp.int32),
            pltpu.SemaphoreType.DMA,
        ],
    )
    def body(a0_hbm, a1_hbm, x32_hbm, cnt_hbm, p0_hbm, p1_hbm, te_hbm,
             sk_hbm, xg_hbm, a0v, a1v, mycnt, cts, p0v, p1v, xrows,
             tev, skv, sem):
        sid = lax.axis_index("s")
        lane = lax.iota(jnp.int32, 16)
        base = sid * tpw

        def splat(v):
            return lax.broadcast_in_dim(jnp.int32(v), (16,), ())

        pltpu.sync_copy(a0_hbm.at[pl.ds(base, tpw)], a0v)
        pltpu.sync_copy(a1_hbm.at[pl.ds(base, tpw)], a1v)

        cnt = jnp.zeros((16,), jnp.int32)
        for c in range(nchunk):
            e0 = a0v[pl.ds(c * _L, _L)]
            e1 = a1v[pl.ds(c * _L, _L)]
            for e in range(E):
                n = (jnp.sum((e0 == e).astype(jnp.int32))
                     + jnp.sum((e1 == e).astype(jnp.int32)))
                cnt = jnp.where(lane == e, cnt + splat(n), cnt)
        mycnt[0, :] = cnt
        pltpu.sync_copy(mycnt, cnt_hbm.at[pl.ds(sid, 1)])
        plsc.subcore_barrier()

        pltpu.sync_copy(cnt_hbm, cts)
        pref = jnp.zeros((16,), jnp.int32)
        gcnt = jnp.zeros((16,), jnp.int32)
        for w in range(_NW):
            row = cts[w, :]
            before = jnp.full((16,), w, jnp.int32) < splat(sid)
            pref = pref + jnp.where(before, row, jnp.zeros((16,), jnp.int32))
            gcnt = gcnt + row
        pc = ((gcnt + jnp.full((16,), TILE - 1, jnp.int32))
             & jnp.full((16,), -TILE, jnp.int32))
        sum7 = jnp.sum(jnp.where(lane < jnp.full((16,), E - 1, jnp.int32), pc, jnp.zeros((16,), jnp.int32)))
        pc = jnp.where(lane == E - 1, splat(n_buf - sum7), pc)
        ends = plsc.cumsum(pc)
        off_we = (ends - pc) + pref

        zv = jnp.zeros((16,), jnp.int32)
        off_s = [jnp.sum(jnp.where(lane == e, off_we, zv)) for e in range(E)]
        ends_s = [jnp.sum(jnp.where(lane == e, ends, zv)) for e in range(E)]
        rc = [jnp.int32(0)] * E

        for r in range(2):
            for cc in range(half // _L):
                for kk, (av, pv) in enumerate(((a0v, p0v), (a1v, p1v))):
                    ev = av[pl.ds(r * half + cc * _L, _L)]
                    p = jnp.zeros((16,), jnp.int32)
                    for e in range(E):
                        m = ev == e
                        mi = m.astype(jnp.int32)
                        rl = plsc.cumsum(mi) - mi
                        p = jnp.where(m, splat(off_s[e] + rc[e]) + rl, p)
                        rc[e] = rc[e] + jnp.sum(mi)
                    pv[r, pl.ds(cc * _L, _L)] = p

        pltpu.sync_copy(p0v, p0_hbm.at[pl.ds(2 * sid, 2)])
        pltpu.sync_copy(p1v, p1_hbm.at[pl.ds(2 * sid, 2)])

        for r in range(2):
            pltpu.sync_copy(x32_hbm.at[pl.ds(base + r * half, half)], xrows)
            pltpu.async_copy(xrows, xg_hbm.at[p0v.at[r]], sem).wait()
            pltpu.async_copy(xrows, xg_hbm.at[p1v.at[r]], sem).wait()

        @pl.when(sid == 0)
        def _():
            gcnt7 = jnp.sum(jnp.where(lane == E - 1, gcnt, jnp.zeros((16,), jnp.int32)))
            true_end = (ends_s[E - 1] - (n_buf - sum7)
                        + ((gcnt7 + (TILE - 1)) & (-TILE)))
            for c in range(nt48 // _L):
                ts = (lane + c * _L) * TILE
                tec = jnp.zeros((16,), jnp.int32)
                for e in range(E):
                    tec = tec + (ts >= splat(ends_s[e])).astype(jnp.int32)
                tec = jnp.minimum(tec, E - 1)
                tev[pl.ds(c * _L, _L)] = tec
                skv[pl.ds(c * _L, _L)] = (ts >= splat(true_end)).astype(jnp.int32)
            pltpu.sync_copy(tev, te_hbm)
            pltpu.sync_copy(skv, sk_hbm)

    _, p0, p1, te48, sk48, xg32 = body(a0, a1, x32)
    pos0 = p0.reshape(T)
    pos1 = p1.reshape(T)
    return pos0, pos1, xg32, te48[:nt], sk48[:nt]


FFC = 1792


def _weights_changed(te_ref, i):
    prev = te_ref[jnp.maximum(i - 1, 0)]
    return (i == 0) | (te_ref[i] != prev)


def _h_body(te_ref, sk_ref, x_ref, w1_ref, w3_ref, h_ref, w1s_ref, w3s_ref):
    i = pl.program_id(1)

    @pl.when((sk_ref[i] == 0) & _weights_changed(te_ref, i))
    def _():
        w1s_ref[...] = w1_ref[0].astype(jnp.bfloat16)
        w3s_ref[...] = w3_ref[0].astype(jnp.bfloat16)

    @pl.when(sk_ref[i] == 0)
    def _():
        x = x_ref[...]
        a = lax.dot_general(x, w1s_ref[...], (((1,), (1,)), ((), ())),
                            preferred_element_type=jnp.float32)
        b = lax.dot_general(x, w3s_ref[...], (((1,), (1,)), ((), ())),
                            preferred_element_type=jnp.float32)
        h_ref[...] = (a * jax.nn.sigmoid(a) * b).astype(jnp.bfloat16)


def _y_body(te_ref, sk_ref, h_ref, w2_ref, y_ref, w2s_ref):
    i = pl.program_id(0)

    @pl.when((sk_ref[i] == 0) & _weights_changed(te_ref, i))
    def _():
        w2s_ref[...] = w2_ref[0].astype(jnp.bfloat16)

    @pl.when(sk_ref[i] == 0)
    def _():
        y_ref[...] = lax.dot_general(h_ref[...], w2s_ref[...],
                                     (((1,), (1,)), ((), ())),
                                     preferred_element_type=jnp.float32)


def _ffn(te, sk, xg, w1, w3, w2, n_buf):
    nt = n_buf // TILE
    nfc = FF // FFC
    h_spec = pltpu.PrefetchScalarGridSpec(
        num_scalar_prefetch=2,
        grid=(nfc, nt),
        in_specs=[
            pl.BlockSpec((TILE, D), lambda j, i, te, sk: (i, 0)),
            pl.BlockSpec((1, FFC, D), lambda j, i, te, sk: (te[i], j, 0)),
            pl.BlockSpec((1, FFC, D), lambda j, i, te, sk: (te[i], j, 0)),
        ],
        out_specs=pl.BlockSpec((TILE, FFC), lambda j, i, te, sk: (i, j)),
        scratch_shapes=[pltpu.VMEM((FFC, D), jnp.bfloat16),
                        pltpu.VMEM((FFC, D), jnp.bfloat16)],
    )
    h = pl.pallas_call(
        _h_body,
        grid_spec=h_spec,
        out_shape=jax.ShapeDtypeStruct((n_buf, FF), jnp.bfloat16),
        interpret=_INTERPRET,
    )(te, sk, xg, w1, w3)
    y_spec = pltpu.PrefetchScalarGridSpec(
        num_scalar_prefetch=2,
        grid=(nt,),
        in_specs=[
            pl.BlockSpec((TILE, FF), lambda i, te, sk: (i, 0)),
            pl.BlockSpec((1, D, FF), lambda i, te, sk: (te[i], 0, 0)),
        ],
        out_specs=pl.BlockSpec((TILE, D), lambda i, te, sk: (i, 0)),
        scratch_shapes=[pltpu.VMEM((D, FF), jnp.bfloat16)],
    )
    return pl.pallas_call(
        _y_body,
        grid_spec=y_spec,
        out_shape=jax.ShapeDtypeStruct((n_buf, D), jnp.float32),
        interpret=_INTERPRET,
    )(te, sk, h, w2)


def kernel(hidden_states, gate_w, w1, w3, w2):
    B, S, _ = hidden_states.shape
    T = B * S
    n_buf = 2 * T + E * TILE
    x = hidden_states.reshape(T, D)

    logits, a0, a1, w0, w1w = _router(x, gate_w)
    a0, a1 = a0[:, 0], a1[:, 0]
    w0, w1w = w0[:, 0], w1w[:, 0]

    xb = x.astype(jnp.bfloat16)
    x32 = lax.bitcast_convert_type(xb.reshape(T, D // 2, 2), jnp.float32)
    pos0, pos1, xg32, te, sk = _sc_dispatch(a0, a1, x32, T, n_buf)
    xg = lax.bitcast_convert_type(xg32, jnp.bfloat16).reshape(n_buf, D)

    y = _ffn(te, sk, xg, w1, w3, w2, n_buf)

    final = (w0[:, None] * jnp.take(y, pos0, axis=0)
             + w1w[:, None] * jnp.take(y, pos1, axis=0))
    return final.reshape(B, S, D), logits


# speedup vs baseline: 1.5743x; 1.5743x over previous
import functools

import jax
import jax.numpy as jnp
from jax import lax
from jax.experimental import pallas as pl
from jax.experimental.pallas import tpu as pltpu
from jax.experimental.pallas import tpu_sc as plsc

E = 8
TOP_K = 2
D = 1024
FF = 3584
TILE = 256

_INTERPRET = False


def _router_body(x_ref, g_ref, logits_ref, a0_ref, a1_ref, w0_ref, w1_ref):
    x = x_ref[...]
    logits = lax.dot_general(x, g_ref[...], (((1,), (1,)), ((), ())),
                             preferred_element_type=jnp.float32)
    logits_ref[...] = logits
    col = lax.broadcasted_iota(jnp.int32, logits.shape, 1)
    m0 = jnp.max(logits, axis=1, keepdims=True)
    is0 = logits == m0
    a0 = jnp.min(jnp.where(is0, col, E), axis=1, keepdims=True)
    masked = jnp.where(col == a0, -jnp.inf, logits)
    m1 = jnp.max(masked, axis=1, keepdims=True)
    a1 = jnp.min(jnp.where(masked == m1, col, E), axis=1, keepdims=True)
    a0_ref[...] = a0
    a1_ref[...] = a1
    w0_ref[...] = jax.nn.sigmoid(m0 - m1)
    w1_ref[...] = jax.nn.sigmoid(m1 - m0)


def _router(x, gate_w):
    T = x.shape[0]
    TB = 512
    grid = (T // TB,)
    out_shapes = (
        jax.ShapeDtypeStruct((T, E), jnp.float32),
        jax.ShapeDtypeStruct((T, 1), jnp.int32),
        jax.ShapeDtypeStruct((T, 1), jnp.int32),
        jax.ShapeDtypeStruct((T, 1), jnp.float32),
        jax.ShapeDtypeStruct((T, 1), jnp.float32),
    )
    row_spec = pl.BlockSpec((TB, 1), lambda i: (i, 0))
    return pl.pallas_call(
        _router_body,
        grid=grid,
        in_specs=[
            pl.BlockSpec((TB, D), lambda i: (i, 0)),
            pl.BlockSpec((E, D), lambda i: (0, 0)),
        ],
        out_specs=(pl.BlockSpec((TB, E), lambda i: (i, 0)),
                   row_spec, row_spec, row_spec, row_spec),
        out_shape=out_shapes,
        interpret=_INTERPRET,
    )(x, gate_w)


_NW = 16
_L = 16


def _sc_dispatch(a0, a1, x, T, n_buf):
    tpw = T // _NW
    nr = 4
    q = tpw // nr
    nchunk = tpw // _L
    nt = n_buf // TILE
    nt48 = 48
    mesh = plsc.VectorSubcoreMesh(core_axis_name="c", subcore_axis_name="s",
                                  num_cores=1)

    @functools.partial(
        pl.kernel, mesh=mesh,
        compiler_params=pltpu.CompilerParams(needs_layout_passes=False),
        out_type=[
            jax.ShapeDtypeStruct((_NW, 16), jnp.int32),
            jax.ShapeDtypeStruct((nr * _NW, q), jnp.int32),
            jax.ShapeDtypeStruct((nr * _NW, q), jnp.int32),
            jax.ShapeDtypeStruct((nt48,), jnp.int32),
            jax.ShapeDtypeStruct((nt48,), jnp.int32),
            jax.ShapeDtypeStruct((n_buf, x.shape[1]), jnp.float32),
        ],
        scratch_types=[
            pltpu.VMEM((tpw,), jnp.int32),
            pltpu.VMEM((tpw,), jnp.int32),
            pltpu.VMEM((1, 16), jnp.int32),
            pltpu.VMEM((_NW, 16), jnp.int32),
            pltpu.VMEM((nr, q), jnp.int32),
            pltpu.VMEM((nr, q), jnp.int32),
            pltpu.VMEM((q, x.shape[1]), jnp.float32),
            pltpu.VMEM((nt48,), jnp.int32),
            pltpu.VMEM((nt48,), jnp.int32),
            pltpu.SemaphoreType.DMA,
        ],
    )
    def body(a0_hbm, a1_hbm, x_hbm, cnt_hbm, p0_hbm, p1_hbm, te_hbm,
             sk_hbm, xg_hbm, a0v, a1v, mycnt, cts, p0v, p1v, xrows,
             tev, skv, sem):
        sid = lax.axis_index("s")
        lane = lax.iota(jnp.int32, 16)
        base = sid * tpw

        def splat(v):
            return lax.broadcast_in_dim(jnp.int32(v), (16,), ())

        pltpu.sync_copy(a0_hbm.at[pl.ds(base, tpw)], a0v)
        pltpu.sync_copy(a1_hbm.at[pl.ds(base, tpw)], a1v)

        cnt = jnp.zeros((16,), jnp.int32)
        for c in range(nchunk):
            e0 = a0v[pl.ds(c * _L, _L)]
            e1 = a1v[pl.ds(c * _L, _L)]
            for e in range(E):
                n = (jnp.sum((e0 == e).astype(jnp.int32))
                     + jnp.sum((e1 == e).astype(jnp.int32)))
                cnt = jnp.where(lane == e, cnt + splat(n), cnt)
        mycnt[0, :] = cnt
        pltpu.sync_copy(mycnt, cnt_hbm.at[pl.ds(sid, 1)])
        plsc.subcore_barrier()

        pltpu.sync_copy(cnt_hbm, cts)
        pref = jnp.zeros((16,), jnp.int32)
        gcnt = jnp.zeros((16,), jnp.int32)
        for w in range(_NW):
            row = cts[w, :]
            before = jnp.full((16,), w, jnp.int32) < splat(sid)
            pref = pref + jnp.where(before, row, jnp.zeros((16,), jnp.int32))
            gcnt = gcnt + row
        pc = ((gcnt + jnp.full((16,), TILE - 1, jnp.int32))
             & jnp.full((16,), -TILE, jnp.int32))
        sum7 = jnp.sum(jnp.where(lane < jnp.full((16,), E - 1, jnp.int32), pc, jnp.zeros((16,), jnp.int32)))
        pc = jnp.where(lane == E - 1, splat(n_buf - sum7), pc)
        ends = plsc.cumsum(pc)
        off_we = (ends - pc) + pref

        zv = jnp.zeros((16,), jnp.int32)
        off_s = [jnp.sum(jnp.where(lane == e, off_we, zv)) for e in range(E)]
        ends_s = [jnp.sum(jnp.where(lane == e, ends, zv)) for e in range(E)]
        rc = [jnp.int32(0)] * E

        for r in range(nr):
            for cc in range(q // _L):
                for kk, (av, pv) in enumerate(((a0v, p0v), (a1v, p1v))):
                    ev = av[pl.ds(r * q + cc * _L, _L)]
                    p = jnp.zeros((16,), jnp.int32)
                    for e in range(E):
                        m = ev == e
                        mi = m.astype(jnp.int32)
                        rl = plsc.cumsum(mi) - mi
                        p = jnp.where(m, splat(off_s[e] + rc[e]) + rl, p)
                        rc[e] = rc[e] + jnp.sum(mi)
                    pv[r, pl.ds(cc * _L, _L)] = p

        pltpu.sync_copy(p0v, p0_hbm.at[pl.ds(nr * sid, nr)])
        pltpu.sync_copy(p1v, p1_hbm.at[pl.ds(nr * sid, nr)])

        for r in range(nr):
            pltpu.sync_copy(x_hbm.at[pl.ds(base + r * q, q)], xrows)
            pltpu.async_copy(xrows, xg_hbm.at[p0v.at[r]], sem).wait()
            pltpu.async_copy(xrows, xg_hbm.at[p1v.at[r]], sem).wait()

        @pl.when(sid == 0)
        def _():
            gcnt7 = jnp.sum(jnp.where(lane == E - 1, gcnt, jnp.zeros((16,), jnp.int32)))
            true_end = (ends_s[E - 1] - (n_buf - sum7)
                        + ((gcnt7 + (TILE - 1)) & (-TILE)))
            for c in range(nt48 // _L):
                ts = (lane + c * _L) * TILE
                tec = jnp.zeros((16,), jnp.int32)
                for e in range(E):
                    tec = tec + (ts >= splat(ends_s[e])).astype(jnp.int32)
                tec = jnp.minimum(tec, E - 1)
                tev[pl.ds(c * _L, _L)] = tec
                skv[pl.ds(c * _L, _L)] = (ts >= splat(true_end)).astype(jnp.int32)
            pltpu.sync_copy(tev, te_hbm)
            pltpu.sync_copy(skv, sk_hbm)

    _, p0, p1, te48, sk48, xg = body(a0, a1, x)
    return p0.reshape(T), p1.reshape(T), xg, te48[:nt], sk48[:nt]


FFC = 1792


def _weights_changed(te_ref, i):
    prev = te_ref[jnp.maximum(i - 1, 0)]
    return (i == 0) | (te_ref[i] != prev)


def _h_body(te_ref, sk_ref, x_ref, w1_ref, w3_ref, h_ref, w1s_ref, w3s_ref):
    i = pl.program_id(1)

    @pl.when((sk_ref[i] == 0) & _weights_changed(te_ref, i))
    def _():
        w1s_ref[...] = w1_ref[0].astype(jnp.bfloat16)
        w3s_ref[...] = w3_ref[0].astype(jnp.bfloat16)

    @pl.when(sk_ref[i] == 0)
    def _():
        x = x_ref[...].astype(jnp.bfloat16)
        a = lax.dot_general(x, w1s_ref[...], (((1,), (1,)), ((), ())),
                            preferred_element_type=jnp.float32)
        b = lax.dot_general(x, w3s_ref[...], (((1,), (1,)), ((), ())),
                            preferred_element_type=jnp.float32)
        h_ref[...] = (a * jax.nn.sigmoid(a) * b).astype(jnp.bfloat16)


def _y_body(te_ref, sk_ref, h_ref, w2_ref, y_ref, w2s_ref):
    i = pl.program_id(0)

    @pl.when((sk_ref[i] == 0) & _weights_changed(te_ref, i))
    def _():
        w2s_ref[...] = w2_ref[0].astype(jnp.bfloat16)

    @pl.when(sk_ref[i] == 0)
    def _():
        y_ref[...] = lax.dot_general(h_ref[...], w2s_ref[...],
                                     (((1,), (1,)), ((), ())),
                                     preferred_element_type=jnp.float32)


def _ffn(te, sk, xg, w1, w3, w2, n_buf):
    nt = n_buf // TILE
    nfc = FF // FFC
    h_spec = pltpu.PrefetchScalarGridSpec(
        num_scalar_prefetch=2,
        grid=(nfc, nt),
        in_specs=[
            pl.BlockSpec((TILE, D), lambda j, i, te, sk: (i, 0)),
            pl.BlockSpec((1, FFC, D), lambda j, i, te, sk: (te[i], j, 0)),
            pl.BlockSpec((1, FFC, D), lambda j, i, te, sk: (te[i], j, 0)),
        ],
        out_specs=pl.BlockSpec((TILE, FFC), lambda j, i, te, sk: (i, j)),
        scratch_shapes=[pltpu.VMEM((FFC, D), jnp.bfloat16),
                        pltpu.VMEM((FFC, D), jnp.bfloat16)],
    )
    h = pl.pallas_call(
        _h_body,
        grid_spec=h_spec,
        out_shape=jax.ShapeDtypeStruct((n_buf, FF), jnp.bfloat16),
        interpret=_INTERPRET,
    )(te, sk, xg, w1, w3)
    y_spec = pltpu.PrefetchScalarGridSpec(
        num_scalar_prefetch=2,
        grid=(nt,),
        in_specs=[
            pl.BlockSpec((TILE, FF), lambda i, te, sk: (i, 0)),
            pl.BlockSpec((1, D, FF), lambda i, te, sk: (te[i], 0, 0)),
        ],
        out_specs=pl.BlockSpec((TILE, D), lambda i, te, sk: (i, 0)),
        scratch_shapes=[pltpu.VMEM((D, FF), jnp.bfloat16)],
    )
    return pl.pallas_call(
        _y_body,
        grid_spec=y_spec,
        out_shape=jax.ShapeDtypeStruct((n_buf, D), jnp.float32),
        interpret=_INTERPRET,
    )(te, sk, h, w2)


def kernel(hidden_states, gate_w, w1, w3, w2):
    B, S, _ = hidden_states.shape
    T = B * S
    n_buf = 2 * T + E * TILE
    x = hidden_states.reshape(T, D)

    logits, a0, a1, w0, w1w = _router(x, gate_w)
    a0, a1 = a0[:, 0], a1[:, 0]
    w0, w1w = w0[:, 0], w1w[:, 0]

    pos0, pos1, xg, te, sk = _sc_dispatch(a0, a1, x, T, n_buf)

    y = _ffn(te, sk, xg, w1, w3, w2, n_buf)

    final = (w0[:, None] * jnp.take(y, pos0, axis=0)
             + w1w[:, None] * jnp.take(y, pos1, axis=0))
    return final.reshape(B, S, D), logits
